# baseline (speedup 1.0000x reference)
import functools

import jax
import jax.numpy as jnp
from jax import lax
from jax.experimental import pallas as pl
from jax.experimental.pallas import tpu as pltpu
from jax.experimental.pallas import tpu_sc as plsc

N = 10000
E = 320000
D = 128
NC = 2
NS = 16
NW = NC * NS
CH = 64
NCHUNK = 157
E_PAD = NW * NCHUNK * CH
DEN_ROWS = 640
NP = DEN_ROWS * 16
ROWS_PER_TILE = NP // NS
LRELU = 0.2
EPS = 1e-16


BR = 512


def _write_asad(asad_ref, h, av):
    i = pl.program_id(0)
    sl = pl.ds(i * BR, BR)
    asad_ref[0, sl] = jnp.sum(h * av[0][None, :], axis=1)
    asad_ref[1, sl] = jnp.sum(h * av[1][None, :], axis=1)


def _tc_first_body(x_ref, w_ref, av_ref, h_ref, asad_ref):
    h = jnp.dot(x_ref[...], w_ref[...], preferred_element_type=jnp.float32)
    h_ref[...] = h
    _write_asad(asad_ref, h, av_ref[...])


def _norm_relu(acc_ref, den_ref, b_ref):
    i = pl.program_id(0)
    sl = pl.ds(i * BR, BR)
    a = acc_ref[0] + acc_ref[1]
    dn = den_ref[0, sl] + den_ref[1, sl]
    xb = a / (dn[:, None] + EPS) + b_ref[...]
    return jnp.maximum(xb, 0.0)


def _tc_mid_body(acc_ref, den_ref, b_ref, w_ref, av_ref, h_ref, asad_ref):
    xb = _norm_relu(acc_ref, den_ref, b_ref)
    h = jnp.dot(xb, w_ref[...], preferred_element_type=jnp.float32)
    h_ref[...] = h
    _write_asad(asad_ref, h, av_ref[...])


def _tc_last_body(acc_ref, den_ref, b_ref, wl_ref, bl_ref, out_ref):
    xb = _norm_relu(acc_ref, den_ref, b_ref)
    out_ref[...] = (
        jnp.dot(xb, wl_ref[...], preferred_element_type=jnp.float32)
        + bl_ref[...]
    )


_H_OUT = [
    jax.ShapeDtypeStruct((NP, D), jnp.float32),
    jax.ShapeDtypeStruct((2, NP), jnp.float32),
]
_H_OUT_SPECS = [
    pl.BlockSpec((BR, D), lambda i: (i, 0)),
    pl.BlockSpec((2, NP), lambda i: (0, 0)),
]


def _tc_first(x, w, av):
    return pl.pallas_call(
        _tc_first_body,
        grid=(NP // BR,),
        in_specs=[
            pl.BlockSpec((BR, D), lambda i: (i, 0)),
            pl.BlockSpec((D, D), lambda i: (0, 0)),
            pl.BlockSpec((2, D), lambda i: (0, 0)),
        ],
        out_specs=_H_OUT_SPECS,
        out_shape=_H_OUT,
    )(x, w, av)


def _tc_mid(acc, den, b, w, av):
    return pl.pallas_call(
        _tc_mid_body,
        grid=(NP // BR,),
        in_specs=[
            pl.BlockSpec((2, BR, D), lambda i: (0, i, 0)),
            pl.BlockSpec((2, NP), lambda i: (0, 0)),
            pl.BlockSpec((1, D), lambda i: (0, 0)),
            pl.BlockSpec((D, D), lambda i: (0, 0)),
            pl.BlockSpec((2, D), lambda i: (0, 0)),
        ],
        out_specs=_H_OUT_SPECS,
        out_shape=_H_OUT,
    )(acc, den, b, w, av)


def _tc_last(acc, den, b, wl, bl):
    return pl.pallas_call(
        _tc_last_body,
        grid=(NP // BR,),
        in_specs=[
            pl.BlockSpec((2, BR, D), lambda i: (0, i, 0)),
            pl.BlockSpec((2, NP), lambda i: (0, 0)),
            pl.BlockSpec((1, D), lambda i: (0, 0)),
            pl.BlockSpec((D, D), lambda i: (0, 0)),
            pl.BlockSpec((1, D), lambda i: (0, 0)),
        ],
        out_specs=pl.BlockSpec((BR, D), lambda i: (i, 0)),
        out_shape=jax.ShapeDtypeStruct((NP, D), jnp.float32),
    )(acc, den, b, wl, bl)


def _sc_edge_body(
    h_hbm, as_hbm, ad_hbm, epk_hbm,
    acc_hbm, den_hbm,
    e_idx, sidx, asb, adb, gbuf, exb,
    denl, idxl,
    acc_sh, den_sh,
    isem0, isem1, isem2, isem3,
    gsem0, gsem1, gsem2, gsem3,
    ssem0, ssem1, ssem2, ssem3,
):
    c = lax.axis_index("c")
    s = lax.axis_index("s")
    w = c * NS + s
    rbase = s * ROWS_PER_TILE
    zeros16 = jnp.zeros((16,), jnp.float32)
    isems = (isem0, isem1, isem2, isem3)
    gsems = (gsem0, gsem1, gsem2, gsem3)
    ssems = (ssem0, ssem1, ssem2, ssem3)

    for _b in range(4):
        pltpu.make_async_copy(
            epk_hbm.at[w].at[_b], e_idx.at[_b], isems[_b]
        ).start()

    def _z_denl(i, _):
        denl[i] = zeros16
        return 0
    lax.fori_loop(0, DEN_ROWS, _z_denl, 0)

    def _z_idx(i, _):
        idxl[pl.ds(i * 16, 16)] = lax.iota(jnp.int32, 16) + i * 16
        return 0
    lax.fori_loop(0, DEN_ROWS // 16, _z_idx, 0)

    def _z_gbuf(r, _):
        for jj in range(D // 16):
            gbuf[0, r, pl.ds(jj * 16, 16)] = zeros16
        return 0
    lax.fori_loop(0, CH, _z_gbuf, 0)

    for k in range(ROWS_PER_TILE // CH):
        pltpu.sync_copy(gbuf.at[0], acc_sh.at[pl.ds(rbase + k * CH, CH)])

    @pl.when(s == 0)
    def _():
        pltpu.sync_copy(denl, den_sh)

    plsc.subcore_barrier()

    def idx_dma(ci, b):
        return pltpu.make_async_copy(
            epk_hbm.at[w].at[ci], e_idx.at[b], isems[b]
        )

    def gathers(ci, b):
        gs = (
            pltpu.make_async_copy(
                as_hbm.at[e_idx.at[b, 0]], asb.at[b], gsems[b]
            ),
            pltpu.make_async_copy(
                ad_hbm.at[e_idx.at[b, 1]], adb.at[b], gsems[b]
            ),
            pltpu.make_async_copy(
                h_hbm.at[e_idx.at[b, 0]], gbuf.at[b], gsems[b]
            ),
        )
        return gs

    def start_gathers(ci, b):
        for g in gathers(ci, b):
            g.start()

    def wait_gathers(ci, b):
        for g in gathers(ci, b):
            g.wait()

    def scatter(ci, b):
        return pltpu.async_copy(
            gbuf.at[b], acc_sh.at[sidx.at[b]], ssems[b], add=True
        )

    def wait_scatter(b):
        pltpu.make_async_copy(
            gbuf.at[b], acc_sh.at[sidx.at[b]], ssems[b]
        ).wait()

    def compute(b):
        for k in range(CH // 16):
            sl = pl.ds(k * 16, 16)
            d16 = e_idx[b, 1, sl]
            al = asb[b, sl] + adb[b, sl]
            al = jnp.where(al >= 0.0, al, al * LRELU)
            ex16 = jnp.exp(al)
            exb[sl] = ex16
            sidx[b, sl] = d16
            plsc.addupdate_scatter(
                denl,
                [lax.shift_right_logical(d16, 4), jnp.bitwise_and(d16, 15)],
                ex16,
            )

        def row(r, _):
            e16 = plsc.load_gather(exb, [jnp.full((16,), r, jnp.int32)])
            for jj in range(D // 16):
                sl = pl.ds(jj * 16, 16)
                gbuf[b, r, sl] = gbuf[b, r, sl] * e16
            return 0
        lax.fori_loop(0, CH, row, 0, unroll=8)

    def step(ci, b):
        wait_gathers(ci, b)
        compute(b)

        @pl.when(ci + 4 < NCHUNK)
        def _():
            idx_dma(ci + 4, b).start()

        scatter(ci, b)
        bg = (b + 3) % 4

        @pl.when(ci + 3 < NCHUNK)
        def _():
            idx_dma(ci + 3, bg).wait()
            wait_scatter(bg)
            start_gathers(ci + 3, bg)

    for _b in range(3):
        idx_dma(_b, _b).wait()
        start_gathers(_b, _b)

    wait_gathers(0, 0)
    compute(0)
    idx_dma(4, 0).start()
    scatter(0, 0)
    idx_dma(3, 3).wait()
    start_gathers(3, 3)

    def iter4(j, _):
        c = 4 * j + 1
        step(c, 1)
        step(c + 1, 2)
        step(c + 2, 3)
        step(c + 3, 0)
        return 0

    lax.fori_loop(0, (NCHUNK - 1) // 4, iter4, 0)

    wait_scatter(0)
    wait_scatter(1)
    wait_scatter(2)
    wait_scatter(3)

    pltpu.sync_copy(denl, den_sh.at[idxl], add=True)
    plsc.subcore_barrier()

    pltpu.sync_copy(
        acc_sh.at[pl.ds(rbase, ROWS_PER_TILE)],
        acc_hbm.at[c].at[pl.ds(rbase, ROWS_PER_TILE)],
    )
    dstripe = DEN_ROWS // NS
    pltpu.sync_copy(
        den_sh.at[pl.ds(s * dstripe, dstripe)],
        den_hbm.at[c].at[pl.ds(s * dstripe, dstripe)],
    )


@functools.lru_cache(maxsize=1)
def _sc_edge_call():
    mesh = plsc.VectorSubcoreMesh(
        core_axis_name="c", subcore_axis_name="s",
        num_cores=NC, num_subcores=NS,
    )
    return pl.kernel(
        _sc_edge_body,
        out_type=[
            jax.ShapeDtypeStruct((NC, NP, D), jnp.float32),
            jax.ShapeDtypeStruct((NC, DEN_ROWS, 16), jnp.float32),
        ],
        mesh=mesh,
        compiler_params=pltpu.CompilerParams(
            needs_layout_passes=False, use_tc_tiling_on_sc=False
        ),
        scratch_types=[
            pltpu.VMEM((4, 2, CH), jnp.int32),
            pltpu.VMEM((4, CH), jnp.int32),
            pltpu.VMEM((4, CH), jnp.float32),
            pltpu.VMEM((4, CH), jnp.float32),
            pltpu.VMEM((4, CH, D), jnp.float32),
            pltpu.VMEM((CH,), jnp.float32),
            pltpu.VMEM((DEN_ROWS, 16), jnp.float32),
            pltpu.VMEM((DEN_ROWS,), jnp.int32),
            pltpu.VMEM_SHARED((NP, D), jnp.float32),
            pltpu.VMEM_SHARED((DEN_ROWS, 16), jnp.float32),
        ] + [pltpu.SemaphoreType.DMA] * 12,
    )


def kernel(x, edge_index, W1, a_src1, a_dst1, b1, W2, a_src2, a_dst2, b2,
           W3, a_src3, a_dst3, b3, Wl, bl):
    src_pad = jnp.pad(edge_index[0], (0, E_PAD - E)).reshape(NW, NCHUNK, CH)
    dst_pad = jnp.pad(
        edge_index[1], (0, E_PAD - E), constant_values=N
    ).reshape(NW, NCHUNK, CH)
    epk = jnp.stack([src_pad, dst_pad], axis=2)
    sc_edge = _sc_edge_call()

    def layer(h, as_a, ad_a):
        acc, den = sc_edge(h, as_a, ad_a, epk)
        return acc, den.reshape(NC, NP)

    av1 = jnp.concatenate([a_src1, a_dst1], axis=0)
    av2 = jnp.concatenate([a_src2, a_dst2], axis=0)
    av3 = jnp.concatenate([a_src3, a_dst3], axis=0)

    x_pad = jnp.pad(x, ((0, NP - N), (0, 0)))
    h1, asad1 = _tc_first(x_pad, W1, av1)
    acc1, den1 = layer(h1, asad1[0], asad1[1])
    h2, asad2 = _tc_mid(acc1, den1, b1.reshape(1, D), W2, av2)
    acc2, den2 = layer(h2, asad2[0], asad2[1])
    h3, asad3 = _tc_mid(acc2, den2, b2.reshape(1, D), W3, av3)
    acc3, den3 = layer(h3, asad3[0], asad3[1])
    out = _tc_last(acc3, den3, b3.reshape(1, D), Wl, bl.reshape(1, D))
    return out[:N]


# speedup vs baseline: 1.2219x; 1.2219x over previous
import functools

import jax
import jax.numpy as jnp
from jax import lax
from jax.experimental import pallas as pl
from jax.experimental.pallas import tpu as pltpu
from jax.experimental.pallas import tpu_sc as plsc

N = 10000
E = 320000
D = 128
NC = 2
NS = 16
NW = NC * NS
CH = 80
NCHUNK = 125
E_PAD = NW * NCHUNK * CH
DEN_ROWS = 640
NP = DEN_ROWS * 16
ROWS_PER_TILE = NP // NS
LRELU = 0.2
EPS = 1e-16


BR = 512


def _write_asad(asad_ref, h, av):
    i = pl.program_id(0)
    sl = pl.ds(i * BR, BR)
    asad_ref[0, sl] = jnp.sum(h * av[0][None, :], axis=1)
    asad_ref[1, sl] = jnp.sum(h * av[1][None, :], axis=1)


def _tc_first_body(x_ref, w_ref, av_ref, h_ref, asad_ref):
    h = jnp.dot(x_ref[...], w_ref[...], preferred_element_type=jnp.float32)
    h_ref[...] = h
    _write_asad(asad_ref, h, av_ref[...])


def _norm_relu(acc_ref, den_ref, b_ref):
    i = pl.program_id(0)
    sl = pl.ds(i * BR, BR)
    a = acc_ref[0] + acc_ref[1]
    dn = den_ref[0, sl] + den_ref[1, sl]
    xb = a / (dn[:, None] + EPS) + b_ref[...]
    return jnp.maximum(xb, 0.0)


def _tc_mid_body(acc_ref, den_ref, b_ref, w_ref, av_ref, h_ref, asad_ref):
    xb = _norm_relu(acc_ref, den_ref, b_ref)
    h = jnp.dot(xb, w_ref[...], preferred_element_type=jnp.float32)
    h_ref[...] = h
    _write_asad(asad_ref, h, av_ref[...])


def _tc_last_body(acc_ref, den_ref, b_ref, wl_ref, bl_ref, out_ref):
    xb = _norm_relu(acc_ref, den_ref, b_ref)
    out_ref[...] = (
        jnp.dot(xb, wl_ref[...], preferred_element_type=jnp.float32)
        + bl_ref[...]
    )


_H_OUT = [
    jax.ShapeDtypeStruct((NP, D), jnp.float32),
    jax.ShapeDtypeStruct((2, NP), jnp.float32),
]
_H_OUT_SPECS = [
    pl.BlockSpec((BR, D), lambda i: (i, 0)),
    pl.BlockSpec((2, NP), lambda i: (0, 0)),
]


def _tc_first(x, w, av):
    return pl.pallas_call(
        _tc_first_body,
        grid=(NP // BR,),
        in_specs=[
            pl.BlockSpec((BR, D), lambda i: (i, 0)),
            pl.BlockSpec((D, D), lambda i: (0, 0)),
            pl.BlockSpec((2, D), lambda i: (0, 0)),
        ],
        out_specs=_H_OUT_SPECS,
        out_shape=_H_OUT,
    )(x, w, av)


def _tc_mid(acc, den, b, w, av):
    return pl.pallas_call(
        _tc_mid_body,
        grid=(NP // BR,),
        in_specs=[
            pl.BlockSpec((2, BR, D), lambda i: (0, i, 0)),
            pl.BlockSpec((2, NP), lambda i: (0, 0)),
            pl.BlockSpec((1, D), lambda i: (0, 0)),
            pl.BlockSpec((D, D), lambda i: (0, 0)),
            pl.BlockSpec((2, D), lambda i: (0, 0)),
        ],
        out_specs=_H_OUT_SPECS,
        out_shape=_H_OUT,
    )(acc, den, b, w, av)


def _tc_last(acc, den, b, wl, bl):
    return pl.pallas_call(
        _tc_last_body,
        grid=(NP // BR,),
        in_specs=[
            pl.BlockSpec((2, BR, D), lambda i: (0, i, 0)),
            pl.BlockSpec((2, NP), lambda i: (0, 0)),
            pl.BlockSpec((1, D), lambda i: (0, 0)),
            pl.BlockSpec((D, D), lambda i: (0, 0)),
            pl.BlockSpec((1, D), lambda i: (0, 0)),
        ],
        out_specs=pl.BlockSpec((BR, D), lambda i: (i, 0)),
        out_shape=jax.ShapeDtypeStruct((NP, D), jnp.float32),
    )(acc, den, b, wl, bl)


def _sc_edge_body(
    h_hbm, as_hbm, ad_hbm, epk_hbm,
    acc_hbm, den_hbm,
    e_idx, sidx, asb, adb, gbuf, exb,
    denl, idxl,
    acc_sh, den_sh,
    isem0, isem1, isem2, gsem0, gsem1, gsem2,
    ssem0, ssem1, ssem2,
):
    c = lax.axis_index("c")
    s = lax.axis_index("s")
    w = c * NS + s
    rbase = s * ROWS_PER_TILE
    zeros16 = jnp.zeros((16,), jnp.float32)
    isems = (isem0, isem1, isem2)
    gsems = (gsem0, gsem1, gsem2)
    ssems = (ssem0, ssem1, ssem2)

    pltpu.make_async_copy(epk_hbm.at[w].at[0], e_idx.at[0], isem0).start()
    pltpu.make_async_copy(epk_hbm.at[w].at[1], e_idx.at[1], isem1).start()
    pltpu.make_async_copy(epk_hbm.at[w].at[2], e_idx.at[2], isem2).start()

    def _z_denl(i, _):
        denl[i] = zeros16
        return 0
    lax.fori_loop(0, DEN_ROWS, _z_denl, 0)

    def _z_idx(i, _):
        idxl[pl.ds(i * 16, 16)] = lax.iota(jnp.int32, 16) + i * 16
        return 0
    lax.fori_loop(0, DEN_ROWS // 16, _z_idx, 0)

    def _z_gbuf(r, _):
        for jj in range(D // 16):
            gbuf[0, r, pl.ds(jj * 16, 16)] = zeros16
        return 0
    lax.fori_loop(0, CH, _z_gbuf, 0)

    for k in range(ROWS_PER_TILE // CH):
        pltpu.sync_copy(gbuf.at[0], acc_sh.at[pl.ds(rbase + k * CH, CH)])

    @pl.when(s == 0)
    def _():
        pltpu.sync_copy(denl, den_sh)

    plsc.subcore_barrier()

    def idx_dma(ci, b):
        return pltpu.make_async_copy(
            epk_hbm.at[w].at[ci], e_idx.at[b], isems[b]
        )

    def gathers(ci, b):
        gs = (
            pltpu.make_async_copy(
                as_hbm.at[e_idx.at[b, 0]], asb.at[b], gsems[b]
            ),
            pltpu.make_async_copy(
                ad_hbm.at[e_idx.at[b, 1]], adb.at[b], gsems[b]
            ),
            pltpu.make_async_copy(
                h_hbm.at[e_idx.at[b, 0]], gbuf.at[b], gsems[b]
            ),
        )
        return gs

    def start_gathers(ci, b):
        for g in gathers(ci, b):
            g.start()

    def wait_gathers(ci, b):
        for g in gathers(ci, b):
            g.wait()

    def scatter(ci, b):
        return pltpu.async_copy(
            gbuf.at[b], acc_sh.at[sidx.at[b]], ssems[b], add=True
        )

    def wait_scatter(b):
        pltpu.make_async_copy(
            gbuf.at[b], acc_sh.at[sidx.at[b]], ssems[b]
        ).wait()

    def compute(b):
        for k in range(CH // 16):
            sl = pl.ds(k * 16, 16)
            d16 = e_idx[b, 1, sl]
            al = asb[b, sl] + adb[b, sl]
            al = jnp.where(al >= 0.0, al, al * LRELU)
            ex16 = jnp.exp(al)
            exb[sl] = ex16
            sidx[b, sl] = d16
            plsc.addupdate_scatter(
                denl,
                [lax.shift_right_logical(d16, 4), jnp.bitwise_and(d16, 15)],
                ex16,
            )

        def row(r, _):
            e16 = plsc.load_gather(exb, [jnp.full((16,), r, jnp.int32)])
            for jj in range(D // 16):
                sl = pl.ds(jj * 16, 16)
                gbuf[b, r, sl] = gbuf[b, r, sl] * e16
            return 0
        lax.fori_loop(0, CH, row, 0, unroll=8)

    def step(ci, b):
        wait_gathers(ci, b)
        compute(b)

        @pl.when(ci + 3 < NCHUNK)
        def _():
            idx_dma(ci + 3, b).start()

        scatter(ci, b)
        b2 = (b + 2) % 3

        @pl.when(ci + 2 < NCHUNK)
        def _():
            idx_dma(ci + 2, b2).wait()
            wait_scatter(b2)
            start_gathers(ci + 2, b2)

    idx_dma(0, 0).wait()
    start_gathers(0, 0)
    idx_dma(1, 1).wait()
    start_gathers(1, 1)

    wait_gathers(0, 0)
    compute(0)
    idx_dma(3, 0).start()
    scatter(0, 0)
    idx_dma(2, 2).wait()
    start_gathers(2, 2)

    def iter3(j, _):
        c = 3 * j + 1
        step(c, 1)
        step(c + 1, 2)
        step(c + 2, 0)
        return 0

    lax.fori_loop(0, (NCHUNK - 2) // 3, iter3, 0)

    step(NCHUNK - 1, 1)
    wait_scatter(2)
    wait_scatter(0)
    wait_scatter(1)

    pltpu.sync_copy(denl, den_sh.at[idxl], add=True)
    plsc.subcore_barrier()

    pltpu.sync_copy(
        acc_sh.at[pl.ds(rbase, ROWS_PER_TILE)],
        acc_hbm.at[c].at[pl.ds(rbase, ROWS_PER_TILE)],
    )
    dstripe = DEN_ROWS // NS
    pltpu.sync_copy(
        den_sh.at[pl.ds(s * dstripe, dstripe)],
        den_hbm.at[c].at[pl.ds(s * dstripe, dstripe)],
    )


@functools.lru_cache(maxsize=1)
def _sc_edge_call():
    mesh = plsc.VectorSubcoreMesh(
        core_axis_name="c", subcore_axis_name="s",
        num_cores=NC, num_subcores=NS,
    )
    return pl.kernel(
        _sc_edge_body,
        out_type=[
            jax.ShapeDtypeStruct((NC, NP, D), jnp.float32),
            jax.ShapeDtypeStruct((NC, DEN_ROWS, 16), jnp.float32),
        ],
        mesh=mesh,
        compiler_params=pltpu.CompilerParams(
            needs_layout_passes=False, use_tc_tiling_on_sc=False
        ),
        scratch_types=[
            pltpu.VMEM((3, 2, CH), jnp.int32),
            pltpu.VMEM((3, CH), jnp.int32),
            pltpu.VMEM((3, CH), jnp.float32),
            pltpu.VMEM((3, CH), jnp.float32),
            pltpu.VMEM((3, CH, D), jnp.float32),
            pltpu.VMEM((CH,), jnp.float32),
            pltpu.VMEM((DEN_ROWS, 16), jnp.float32),
            pltpu.VMEM((DEN_ROWS,), jnp.int32),
            pltpu.VMEM_SHARED((NP, D), jnp.float32),
            pltpu.VMEM_SHARED((DEN_ROWS, 16), jnp.float32),
        ] + [pltpu.SemaphoreType.DMA] * 9,
    )


def kernel(x, edge_index, W1, a_src1, a_dst1, b1, W2, a_src2, a_dst2, b2,
           W3, a_src3, a_dst3, b3, Wl, bl):
    src_pad = jnp.pad(edge_index[0], (0, E_PAD - E)).reshape(NW, NCHUNK, CH)
    dst_pad = jnp.pad(
        edge_index[1], (0, E_PAD - E), constant_values=N
    ).reshape(NW, NCHUNK, CH)
    epk = jnp.stack([src_pad, dst_pad], axis=2)
    sc_edge = _sc_edge_call()

    def layer(h, as_a, ad_a):
        acc, den = sc_edge(h, as_a, ad_a, epk)
        return acc, den.reshape(NC, NP)

    av1 = jnp.concatenate([a_src1, a_dst1], axis=0)
    av2 = jnp.concatenate([a_src2, a_dst2], axis=0)
    av3 = jnp.concatenate([a_src3, a_dst3], axis=0)

    x_pad = jnp.pad(x, ((0, NP - N), (0, 0)))
    h1, asad1 = _tc_first(x_pad, W1, av1)
    acc1, den1 = layer(h1, asad1[0], asad1[1])
    h2, asad2 = _tc_mid(acc1, den1, b1.reshape(1, D), W2, av2)
    acc2, den2 = layer(h2, asad2[0], asad2[1])
    h3, asad3 = _tc_mid(acc2, den2, b2.reshape(1, D), W3, av3)
    acc3, den3 = layer(h3, asad3[0], asad3[1])
    out = _tc_last(acc3, den3, b3.reshape(1, D), Wl, bl.reshape(1, D))
    return out[:N]
